# Initial kernel scaffold; baseline (speedup 1.0000x reference)
import jax
import jax.numpy as jnp
from jax.experimental import pallas as pl


def kernel(x, edge_index, W_node, b_node, W_edge, b_edge, W_out, b_out):
    raise NotImplementedError("write your pallas kernel here")


# speedup vs baseline: 14.1985x; 14.1985x over previous
import functools

import jax
import jax.numpy as jnp
from jax import lax
from jax.experimental import pallas as pl
from jax.experimental.pallas import tpu as pltpu
from jax.experimental.pallas import tpu_sc as plsc

N = 10000
E = 320000
D = 128
DH = D // 2

NC = 2
NS = 16
NW = NC * NS
EPW = E // NW
EPS = E // NS
CHUNK = 125
NCHUNK = EPS // CHUNK
NPAD = 10240
RPW = NPAD // NS
RCH = 128
LANES = 16

_mesh = plsc.VectorSubcoreMesh(
    core_axis_name="c", subcore_axis_name="s", num_cores=NC, num_subcores=NS
)


def _deg_body(src_hbm, dst_hbm, hist_out, src_v, dst_v, hs_v, hd_v):
    c = lax.axis_index("c")
    s = lax.axis_index("s")
    w = c * NS + s
    pltpu.sync_copy(src_hbm.at[w], src_v)
    pltpu.sync_copy(dst_hbm.at[w], dst_v)
    zero = jnp.zeros((LANES,), jnp.float32)

    @pl.loop(0, NPAD // LANES)
    def _zero(i):
        hs_v[pl.ds(i * LANES, LANES)] = zero
        hd_v[pl.ds(i * LANES, LANES)] = zero

    one = jnp.ones((LANES,), jnp.float32)

    @pl.loop(0, EPW // LANES)
    def _accum(i):
        si = src_v[pl.ds(i * LANES, LANES)]
        di = dst_v[pl.ds(i * LANES, LANES)]
        plsc.addupdate_scatter(hs_v, [si], one)
        plsc.addupdate_scatter(hd_v, [di], one)

    pltpu.sync_copy(hs_v, hist_out.at[w, 0])
    pltpu.sync_copy(hd_v, hist_out.at[w, 1])


_deg_call = pl.kernel(
    _deg_body,
    out_type=jax.ShapeDtypeStruct((NW, 2, NPAD), jnp.float32),
    mesh=_mesh,
    compiler_params=pltpu.CompilerParams(needs_layout_passes=False),
    scratch_types=[
        pltpu.VMEM((EPW,), jnp.int32),
        pltpu.VMEM((EPW,), jnp.int32),
        pltpu.VMEM((NPAD,), jnp.float32),
        pltpu.VMEM((NPAD,), jnp.float32),
    ],
)


def _scatter_body(msg0_hbm, msg1_hbm, srcw_hbm, dstw_hbm, parts_out,
                  src_v, dst_v, rows_v, stage_v, acc_sh, sem):
    c = lax.axis_index("c")
    s = lax.axis_index("s")
    pltpu.sync_copy(srcw_hbm.at[s], src_v)
    pltpu.sync_copy(dstw_hbm.at[s], dst_v)

    zero = jnp.zeros((LANES,), jnp.float32)

    @pl.loop(0, RCH)
    def _zrow(i):
        @pl.loop(0, DH // LANES)
        def _zcol(k):
            stage_v[i, pl.ds(k * LANES, LANES)] = zero

    for k in range(RPW // RCH):
        pltpu.sync_copy(stage_v, acc_sh.at[pl.ds(s * RPW + k * RCH, RCH)])
    plsc.subcore_barrier()

    @pl.when(c == 0)
    def _core0():
        @pl.loop(0, NCHUNK)
        def _edge_chunk(j):
            pltpu.async_copy(msg0_hbm.at[src_v.at[j]], rows_v, sem).wait()
            pltpu.sync_copy(rows_v, acc_sh.at[dst_v.at[j]], add=True)

    @pl.when(c == 1)
    def _core1():
        @pl.loop(0, NCHUNK)
        def _edge_chunk(j):
            pltpu.async_copy(msg1_hbm.at[src_v.at[j]], rows_v, sem).wait()
            pltpu.sync_copy(rows_v, acc_sh.at[dst_v.at[j]], add=True)

    plsc.subcore_barrier()
    for k in range(RPW // RCH):
        pltpu.sync_copy(acc_sh.at[pl.ds(s * RPW + k * RCH, RCH)], stage_v)
        pltpu.sync_copy(stage_v, parts_out.at[c, pl.ds(s * RPW + k * RCH, RCH)])


_scatter_call = pl.kernel(
    _scatter_body,
    out_type=jax.ShapeDtypeStruct((NC, NPAD, DH), jnp.float32),
    mesh=_mesh,
    compiler_params=pltpu.CompilerParams(
        needs_layout_passes=False, use_tc_tiling_on_sc=False
    ),
    scratch_types=[
        pltpu.VMEM((NCHUNK, CHUNK), jnp.int32),
        pltpu.VMEM((NCHUNK, CHUNK), jnp.int32),
        pltpu.VMEM((CHUNK, DH), jnp.float32),
        pltpu.VMEM((RCH, DH), jnp.float32),
        pltpu.VMEM_SHARED((NPAD, DH), jnp.float32),
        pltpu.SemaphoreType.DMA,
    ],
)


def _msg_body(x_ref, wn_ref, bn_ref, we_ref, be_ref, hist_ref,
              out0_ref, out1_ref):
    out_deg = jnp.sum(hist_ref[:, 0, :N], axis=0)
    scale = lax.rsqrt(jnp.maximum(out_deg, 1.0))
    h = jnp.dot(x_ref[...], wn_ref[...], preferred_element_type=jnp.float32)
    h = h + bn_ref[...]
    m = jnp.dot(h, we_ref[...], preferred_element_type=jnp.float32)
    m = m + be_ref[...]
    m = m * scale[:, None]
    out0_ref[...] = m[:, :DH]
    out1_ref[...] = m[:, DH:]


_msg_call = pl.pallas_call(
    _msg_body,
    out_shape=(
        jax.ShapeDtypeStruct((N, DH), jnp.float32),
        jax.ShapeDtypeStruct((N, DH), jnp.float32),
    ),
)


def _out_body(parts_ref, hist_ref, wo_ref, bo_ref, out_ref):
    in_deg = jnp.sum(hist_ref[:, 1, :N], axis=0)
    nrm = lax.rsqrt(jnp.maximum(in_deg, 1.0))
    upd = jnp.concatenate([parts_ref[0, :N, :], parts_ref[1, :N, :]], axis=1)
    upd = upd * nrm[:, None]
    z = jnp.dot(upd, wo_ref[...], preferred_element_type=jnp.float32)
    z = z + bo_ref[...]
    out_ref[...] = z * 0.5 * (1.0 + lax.erf(z * (2.0 ** -0.5)))


_out_call = pl.pallas_call(
    _out_body,
    out_shape=jax.ShapeDtypeStruct((N, D), jnp.float32),
)


def kernel(x, edge_index, W_node, b_node, W_edge, b_edge, W_out, b_out):
    src = edge_index[0].astype(jnp.int32)
    dst = edge_index[1].astype(jnp.int32)

    hist = _deg_call(src.reshape(NW, EPW), dst.reshape(NW, EPW))

    msg0, msg1 = _msg_call(x, W_node, b_node.reshape(1, D), W_edge,
                           b_edge.reshape(1, D), hist)

    srcw = src.reshape(NS, NCHUNK, CHUNK)
    dstw = dst.reshape(NS, NCHUNK, CHUNK)
    parts = _scatter_call(msg0, msg1, srcw, dstw)

    return _out_call(parts, hist, W_out, b_out.reshape(1, D))


# speedup vs baseline: 21.1773x; 1.4915x over previous
import functools

import jax
import jax.numpy as jnp
from jax import lax
from jax.experimental import pallas as pl
from jax.experimental.pallas import tpu as pltpu
from jax.experimental.pallas import tpu_sc as plsc

N = 10000
E = 320000
D = 128
DH = D // 2

NC = 2
NS = 16
NW = NC * NS
EPW = E // NW
EPS = E // NS
CHUNK = 125
NCHUNK = EPS // CHUNK
NPAD = 10240
RPW = NPAD // NS
RCH = 128
LANES = 16

_mesh = plsc.VectorSubcoreMesh(
    core_axis_name="c", subcore_axis_name="s", num_cores=NC, num_subcores=NS
)


def _deg_body(src_hbm, dst_hbm, hist_out, src_v, dst_v, hs_v, hd_v):
    c = lax.axis_index("c")
    s = lax.axis_index("s")
    w = c * NS + s
    pltpu.sync_copy(src_hbm.at[w], src_v)
    pltpu.sync_copy(dst_hbm.at[w], dst_v)
    zero = jnp.zeros((LANES,), jnp.float32)

    @pl.loop(0, NPAD // LANES)
    def _zero(i):
        hs_v[pl.ds(i * LANES, LANES)] = zero
        hd_v[pl.ds(i * LANES, LANES)] = zero

    one = jnp.ones((LANES,), jnp.float32)

    @pl.loop(0, EPW // LANES)
    def _accum(i):
        si = src_v[pl.ds(i * LANES, LANES)]
        di = dst_v[pl.ds(i * LANES, LANES)]
        plsc.addupdate_scatter(hs_v, [si], one)
        plsc.addupdate_scatter(hd_v, [di], one)

    pltpu.sync_copy(hs_v, hist_out.at[w, 0])
    pltpu.sync_copy(hd_v, hist_out.at[w, 1])


_deg_call = pl.kernel(
    _deg_body,
    out_type=jax.ShapeDtypeStruct((NW, 2, NPAD), jnp.float32),
    mesh=_mesh,
    compiler_params=pltpu.CompilerParams(needs_layout_passes=False),
    scratch_types=[
        pltpu.VMEM((EPW,), jnp.int32),
        pltpu.VMEM((EPW,), jnp.int32),
        pltpu.VMEM((NPAD,), jnp.float32),
        pltpu.VMEM((NPAD,), jnp.float32),
    ],
)


def _scatter_body(msg0_hbm, msg1_hbm, srcw_hbm, dstw_hbm, parts_out,
                  src_v, dst_v, rows0_v, rows1_v, stage_v, acc_sh,
                  sem0, sem1):
    c = lax.axis_index("c")
    s = lax.axis_index("s")
    pltpu.sync_copy(srcw_hbm.at[s], src_v)
    pltpu.sync_copy(dstw_hbm.at[s], dst_v)

    zero = jnp.zeros((LANES,), jnp.float32)

    @pl.loop(0, RCH)
    def _zrow(i):
        @pl.loop(0, DH // LANES)
        def _zcol(k):
            stage_v[i, pl.ds(k * LANES, LANES)] = zero

    for k in range(RPW // RCH):
        pltpu.sync_copy(stage_v, acc_sh.at[pl.ds(s * RPW + k * RCH, RCH)])
    plsc.subcore_barrier()

    def _edge_loop(msg_hbm):
        pltpu.async_copy(msg_hbm.at[src_v.at[0]], rows0_v, sem0)
        pltpu.async_copy(msg_hbm.at[src_v.at[1]], rows1_v, sem1)

        @pl.loop(0, NCHUNK // 2)
        def _edge_chunk(jj):
            j = jj * 2
            pltpu.make_async_copy(msg_hbm.at[src_v.at[j]], rows0_v,
                                  sem0).wait()
            pltpu.sync_copy(rows0_v, acc_sh.at[dst_v.at[j]], add=True)

            @pl.when(j + 2 < NCHUNK)
            def _next0():
                pltpu.async_copy(msg_hbm.at[src_v.at[j + 2]], rows0_v, sem0)

            pltpu.make_async_copy(msg_hbm.at[src_v.at[j + 1]], rows1_v,
                                  sem1).wait()
            pltpu.sync_copy(rows1_v, acc_sh.at[dst_v.at[j + 1]], add=True)

            @pl.when(j + 3 < NCHUNK)
            def _next1():
                pltpu.async_copy(msg_hbm.at[src_v.at[j + 3]], rows1_v, sem1)

    @pl.when(c == 0)
    def _core0():
        _edge_loop(msg0_hbm)

    @pl.when(c == 1)
    def _core1():
        _edge_loop(msg1_hbm)

    plsc.subcore_barrier()
    for k in range(RPW // RCH):
        pltpu.sync_copy(acc_sh.at[pl.ds(s * RPW + k * RCH, RCH)], stage_v)
        pltpu.sync_copy(stage_v, parts_out.at[c, pl.ds(s * RPW + k * RCH, RCH)])


_scatter_call = pl.kernel(
    _scatter_body,
    out_type=jax.ShapeDtypeStruct((NC, NPAD, DH), jnp.float32),
    mesh=_mesh,
    compiler_params=pltpu.CompilerParams(
        needs_layout_passes=False, use_tc_tiling_on_sc=False
    ),
    scratch_types=[
        pltpu.VMEM((NCHUNK, CHUNK), jnp.int32),
        pltpu.VMEM((NCHUNK, CHUNK), jnp.int32),
        pltpu.VMEM((CHUNK, DH), jnp.float32),
        pltpu.VMEM((CHUNK, DH), jnp.float32),
        pltpu.VMEM((RCH, DH), jnp.float32),
        pltpu.VMEM_SHARED((NPAD, DH), jnp.float32),
        pltpu.SemaphoreType.DMA,
        pltpu.SemaphoreType.DMA,
    ],
)


def _msg_body(x_ref, wn_ref, bn_ref, we_ref, be_ref, hist_ref,
              out0_ref, out1_ref):
    out_deg = jnp.sum(hist_ref[:, 0, :N], axis=0)
    scale = lax.rsqrt(jnp.maximum(out_deg, 1.0))
    h = jnp.dot(x_ref[...], wn_ref[...], preferred_element_type=jnp.float32)
    h = h + bn_ref[...]
    m = jnp.dot(h, we_ref[...], preferred_element_type=jnp.float32)
    m = m + be_ref[...]
    m = m * scale[:, None]
    out0_ref[...] = m[:, :DH]
    out1_ref[...] = m[:, DH:]


_msg_call = pl.pallas_call(
    _msg_body,
    out_shape=(
        jax.ShapeDtypeStruct((N, DH), jnp.float32),
        jax.ShapeDtypeStruct((N, DH), jnp.float32),
    ),
)


def _out_body(parts_ref, hist_ref, wo_ref, bo_ref, out_ref):
    in_deg = jnp.sum(hist_ref[:, 1, :N], axis=0)
    nrm = lax.rsqrt(jnp.maximum(in_deg, 1.0))
    upd = jnp.concatenate([parts_ref[0, :N, :], parts_ref[1, :N, :]], axis=1)
    upd = upd * nrm[:, None]
    z = jnp.dot(upd, wo_ref[...], preferred_element_type=jnp.float32)
    z = z + bo_ref[...]
    out_ref[...] = z * 0.5 * (1.0 + lax.erf(z * (2.0 ** -0.5)))


_out_call = pl.pallas_call(
    _out_body,
    out_shape=jax.ShapeDtypeStruct((N, D), jnp.float32),
)


def kernel(x, edge_index, W_node, b_node, W_edge, b_edge, W_out, b_out):
    src = edge_index[0].astype(jnp.int32)
    dst = edge_index[1].astype(jnp.int32)

    hist = _deg_call(src.reshape(NW, EPW), dst.reshape(NW, EPW))

    msg0, msg1 = _msg_call(x, W_node, b_node.reshape(1, D), W_edge,
                           b_edge.reshape(1, D), hist)

    srcw = src.reshape(NS, NCHUNK, CHUNK)
    dstw = dst.reshape(NS, NCHUNK, CHUNK)
    parts = _scatter_call(msg0, msg1, srcw, dstw)

    return _out_call(parts, hist, W_out, b_out.reshape(1, D))
